# baseline (speedup 1.0000x reference)
import jax
import jax.numpy as jnp
from jax import lax
from jax.experimental import pallas as pl
from jax.experimental.pallas import tpu as pltpu

_CHUNK = 200


def _policy_kernel(state_ref, W_embed_ref, b_embed_ref, W1_ref, b1_ref,
                   W2_ref, b2_row_ref, b2_col_ref, W_dec_ref, out_ref,
                   p1f_acc, cacc, csxs, row_scratch, si_ref):
    f32 = jnp.float32
    f = W_embed_ref.shape[0]
    D = W_embed_ref.shape[1]
    N = state_ref.shape[1]
    Wn = N - f
    i = pl.program_id(0)
    nsteps = pl.num_programs(0)
    B = state_ref.shape[0]
    S = B * nsteps
    inv_S = 1.0 / S
    inv_W = 1.0 / Wn
    BIG = jnp.int32(2 * S)

    chunk = state_ref[...]

    x_sc = lax.dot_general(chunk[:, :f], W_embed_ref[...],
                           (((1,), (0,)), ((), ())),
                           preferred_element_type=f32) + b_embed_ref[...]
    p1_part = lax.dot_general(chunk, x_sc, (((0,), (0,)), ((), ())),
                              preferred_element_type=f32)
    c_part = jnp.sum(chunk, axis=0, keepdims=True)
    cs_part = jnp.sum(x_sc, axis=0, keepdims=True)

    @pl.when(i == 0)
    def _init():
        p1f_acc[...] = p1_part
        cacc[...] = c_part
        csxs[...] = cs_part
        row_scratch[...] = state_ref[pl.ds(0, 1), :]
        si_ref[0, 0] = jnp.int32(-1)

    @pl.when(i > 0)
    def _acc():
        p1f_acc[...] += p1_part
        cacc[...] += c_part
        csxs[...] += cs_part

    rs = (jnp.sum(chunk, axis=1, keepdims=True)
          - jnp.sum(chunk[:, :f], axis=1, keepdims=True))
    iota_col = lax.broadcasted_iota(jnp.int32, (B, 1), 0) + i * B
    lm = jnp.min(jnp.where(rs == 0.0, iota_col, BIG))

    @pl.when((si_ref[0, 0] < 0) & (lm < BIG))
    def _capture():
        si_ref[0, 0] = lm
        row_scratch[...] = state_ref[pl.ds(lm - i * B, 1), :]

    @pl.when(i == nsteps - 1)
    def _epilogue():
        P1 = p1f_acc[f:, :]
        agg_w1 = (P1 + csxs[...]) * inv_S
        h_w1 = jnp.maximum(
            lax.dot_general(agg_w1, W1_ref[...], (((1,), (0,)), ((), ())),
                            preferred_element_type=f32) + b1_ref[...], 0.0)

        r1 = jnp.maximum(b1_ref[...], 0.0)

        c_row = 1.0 + cacc[...][:, f:] * inv_S
        t_col = lax.dot_general(W2_ref[...], r1, (((0,), (1,)), ((), ())),
                                preferred_element_type=f32)
        h_w2_T = jnp.maximum(t_col * c_row + b2_col_ref[...], 0.0)

        colsum_hw1 = jnp.sum(h_w1, axis=0, keepdims=True)
        a_row = row_scratch[...][:, f:]
        u1 = lax.dot_general(a_row, h_w1, (((1,), (0,)), ((), ())),
                             preferred_element_type=f32)
        agg_si = (u1 + colsum_hw1) * inv_W
        shift_h = jnp.maximum(
            lax.dot_general(agg_si, W2_ref[...], (((1,), (0,)), ((), ())),
                            preferred_element_type=f32) + b2_row_ref[...],
            0.0)

        v = lax.dot_general(shift_h, W_dec_ref[...], (((1,), (1,)), ((), ())),
                            preferred_element_type=f32)
        scores = lax.dot_general(v, h_w2_T, (((1,), (0,)), ((), ())),
                                 preferred_element_type=f32)

        m = jnp.max(scores, axis=1, keepdims=True)
        e = jnp.exp(scores - m)
        out_ref[...] = e / jnp.sum(e, axis=1, keepdims=True)


def kernel(state, W_embed, b_embed, W1, b1, W2, b2, W_dec, count_shifts,
           shift_features):
    f = W_embed.shape[0]
    S = state.shape[0]
    N = state.shape[1]
    Wn = N - f
    D = W_embed.shape[1]
    nsteps = S // _CHUNK
    full = lambda i: (0, 0)
    out = pl.pallas_call(
        _policy_kernel,
        grid=(nsteps,),
        in_specs=[
            pl.BlockSpec((_CHUNK, N), lambda i: (i, 0)),
            pl.BlockSpec((f, D), full),
            pl.BlockSpec((1, D), full),
            pl.BlockSpec((D, D), full),
            pl.BlockSpec((1, D), full),
            pl.BlockSpec((D, D), full),
            pl.BlockSpec((1, D), full),
            pl.BlockSpec((D, 1), full),
            pl.BlockSpec((D, D), full),
        ],
        out_specs=pl.BlockSpec((1, Wn), full),
        scratch_shapes=[
            pltpu.VMEM((N, D), jnp.float32),
            pltpu.VMEM((1, N), jnp.float32),
            pltpu.VMEM((1, D), jnp.float32),
            pltpu.VMEM((1, N), jnp.float32),
            pltpu.SMEM((1, 1), jnp.int32),
        ],
        out_shape=jax.ShapeDtypeStruct((1, Wn), state.dtype),
    )(state, W_embed, b_embed.reshape(1, D), W1, b1.reshape(1, D),
      W2, b2.reshape(1, D), b2.reshape(D, 1), W_dec)
    return out.reshape(Wn)
